# baseline (speedup 1.0000x reference)
import jax
import jax.numpy as jnp
from jax.experimental import pallas as pl
from jax.experimental.pallas import tpu as pltpu

Q = 8
K = 1024
D = 256
BETA = 0.25


def _rvq_body(x_ref, cb_ref, qout_ref, idx_ref, loss_ref):
    res = x_ref[...]
    r = res.shape[0]
    qacc = jnp.zeros_like(res)
    iota_k = jax.lax.broadcasted_iota(jnp.int32, (r, K), 1)
    idx_rows = []
    loss_rows = []
    for q in range(Q):
        cb = cb_ref[q]
        mm = jax.lax.dot_general(
            res, cb, (((1,), (1,)), ((), ())),
            preferred_element_type=jnp.float32)
        rown = jnp.sum(res * res, axis=1, keepdims=True)
        cnorm = jnp.sum(cb * cb, axis=1)
        dist = rown - 2.0 * mm + cnorm[None, :]
        m = jnp.min(dist, axis=1, keepdims=True)
        cand = jnp.where(dist == m, iota_k, K)
        idx = jnp.min(cand, axis=1)
        idx_rows.append(idx)
        onehot = (iota_k == idx[:, None]).astype(jnp.float32)
        qv = jax.lax.dot_general(
            onehot, cb, (((1,), (0,)), ((), ())),
            preferred_element_type=jnp.float32,
            precision=jax.lax.Precision.HIGHEST)
        diff = qv - res
        loss_rows.append(jnp.sum(diff * diff, axis=0))
        res = res - qv
        qacc = qacc + qv
    qout_ref[...] = qacc
    idx_ref[...] = jnp.stack(idx_rows)
    loss_ref[...] = jnp.stack(loss_rows)[None]


def kernel(x, codebooks):
    b, t, d = x.shape
    n = b * t
    tile = 1024
    nt = n // tile
    xf = x.reshape(n, d)
    qout, idx, loss_parts = pl.pallas_call(
        _rvq_body,
        grid=(nt,),
        in_specs=[
            pl.BlockSpec((tile, d), lambda i: (i, 0)),
            pl.BlockSpec((Q, K, D), lambda i: (0, 0, 0)),
        ],
        out_specs=[
            pl.BlockSpec((tile, d), lambda i: (i, 0)),
            pl.BlockSpec((Q, tile), lambda i: (0, i)),
            pl.BlockSpec((1, Q, D), lambda i: (i, 0, 0)),
        ],
        out_shape=[
            jax.ShapeDtypeStruct((n, d), jnp.float32),
            jax.ShapeDtypeStruct((Q, n), jnp.int32),
            jax.ShapeDtypeStruct((nt, Q, D), jnp.float32),
        ],
        compiler_params=pltpu.CompilerParams(
            dimension_semantics=("parallel",),
        ),
    )(xf, codebooks)
    sums = jnp.sum(loss_parts, axis=(0, 2))
    per_layer = sums / (n * d)
    out_loss = jnp.mean(per_layer + BETA * per_layer)
    return qout.reshape(b, t, d), idx.reshape(Q, b, t), out_loss


# speedup vs baseline: 1.8569x; 1.8569x over previous
import jax
import jax.numpy as jnp
from jax.experimental import pallas as pl
from jax.experimental.pallas import tpu as pltpu

Q = 8
K = 1024
D = 256
BETA = 0.25


def _rvq_body(x_ref, cb_ref, cbs_ref, qout_ref, idx_ref, loss_ref):
    res = x_ref[...]
    r = res.shape[0]
    qacc = jnp.zeros_like(res)
    iota_k = jax.lax.broadcasted_iota(jnp.int32, (r, K), 1)
    idx_rows = []
    loss_rows = []
    for q in range(Q):
        cb = cb_ref[q]
        mm = jax.lax.dot_general(
            res, cb, (((1,), (1,)), ((), ())),
            preferred_element_type=jnp.float32)
        rown = jnp.sum(res * res, axis=1, keepdims=True)
        cnorm = jnp.sum(cb * cb, axis=1)
        dist = rown - 2.0 * mm + cnorm[None, :]
        m = jnp.min(dist, axis=1, keepdims=True)
        cand = jnp.where(dist == m, iota_k, K)
        idx = jnp.min(cand, axis=1)
        idx_rows.append(idx)
        onehot = (cand == idx[:, None]).astype(jnp.bfloat16)
        gdims = (((1,), (0,)), ((), ()))
        qv = (
            (jax.lax.dot_general(onehot, cbs_ref[0, q], gdims,
                                 preferred_element_type=jnp.float32)
             + jax.lax.dot_general(onehot, cbs_ref[1, q], gdims,
                                   preferred_element_type=jnp.float32))
            + jax.lax.dot_general(onehot, cbs_ref[2, q], gdims,
                                  preferred_element_type=jnp.float32)
        )
        diff = qv - res
        loss_rows.append(jnp.sum(diff * diff, axis=0))
        res = res - qv
        qacc = qacc + qv
    qout_ref[...] = qacc
    idx_ref[...] = jnp.stack(idx_rows)
    loss_ref[...] = jnp.stack(loss_rows)[None]


def kernel(x, codebooks):
    b, t, d = x.shape
    n = b * t
    tile = 1024
    nt = n // tile
    xf = x.reshape(n, d)
    cb_hi = codebooks.astype(jnp.bfloat16)
    r1 = codebooks - cb_hi.astype(jnp.float32)
    cb_mid = r1.astype(jnp.bfloat16)
    cb_lo = (r1 - cb_mid.astype(jnp.float32)).astype(jnp.bfloat16)
    cb_split = jnp.stack([cb_hi, cb_mid, cb_lo])
    qout, idx, loss_parts = pl.pallas_call(
        _rvq_body,
        grid=(nt,),
        in_specs=[
            pl.BlockSpec((tile, d), lambda i: (i, 0)),
            pl.BlockSpec((Q, K, D), lambda i: (0, 0, 0)),
            pl.BlockSpec((3, Q, K, D), lambda i: (0, 0, 0, 0)),
        ],
        out_specs=[
            pl.BlockSpec((tile, d), lambda i: (i, 0)),
            pl.BlockSpec((Q, tile), lambda i: (0, i)),
            pl.BlockSpec((1, Q, D), lambda i: (i, 0, 0)),
        ],
        out_shape=[
            jax.ShapeDtypeStruct((n, d), jnp.float32),
            jax.ShapeDtypeStruct((Q, n), jnp.int32),
            jax.ShapeDtypeStruct((nt, Q, D), jnp.float32),
        ],
        compiler_params=pltpu.CompilerParams(
            dimension_semantics=("parallel",),
        ),
    )(xf, codebooks, cb_split)
    sums = jnp.sum(loss_parts, axis=(0, 2))
    per_layer = sums / (n * d)
    out_loss = jnp.mean(per_layer + BETA * per_layer)
    return qout.reshape(b, t, d), idx.reshape(Q, b, t), out_loss
